# baseline (speedup 1.0000x reference)
import functools

import jax
import jax.numpy as jnp
from jax import lax
from jax.experimental import pallas as pl
from jax.experimental.pallas import tpu as pltpu
from jax.experimental.pallas import tpu_sc as plsc

B = 4096
S = 50
D = 512
NTAB = 225
TAB_PAD = 256
SENTINEL = 224

BLK = 256
NW = 32
RPW = B // NW
SFULL = 48


def _tc_body(tab_ref, sc_ref, oy_ref, d0, d1, d2, d3, d4, hr_ref, a48, a49,
             lng, lnb, wst, sb, oyat, dtab, hwt, hb, wt, ib,
             out_ref, t48_ref, t49_ref):
    i = pl.program_id(0)

    @pl.when(i == 0)
    def _():
        out_ref[...] = tab_ref[...]

    @pl.when(i > 0)
    def _():
        x = sc_ref[...]
        mu = jnp.mean(x, axis=-1, keepdims=True)
        xc = x - mu
        var = jnp.mean(xc * xc, axis=-1, keepdims=True)
        xn = xc * lax.rsqrt(var + 1e-5) * lng[...] + lnb[...]
        s_emb = jnp.dot(xn, wst[...], preferred_element_type=jnp.float32) + sb[...]

        oh = (oy_ref[...] == lax.broadcasted_iota(jnp.int32, (BLK, 4), 1))
        oya_emb = jnp.dot(oh.astype(jnp.float32), oyat[...],
                          preferred_element_type=jnp.float32)

        h_emb = jnp.dot(hr_ref[...], hwt[...],
                        preferred_element_type=jnp.float32) + hb[...]

        acc = jnp.dot(s_emb, wt[0:32, :], preferred_element_type=jnp.float32)
        acc += jnp.dot(oya_emb, wt[32:48, :], preferred_element_type=jnp.float32)
        for j, dref in enumerate((d0, d1, d2, d3, d4)):
            ohd = (dref[...] == lax.broadcasted_iota(jnp.int32, (BLK, 38), 1))
            dora_emb = jnp.dot(ohd.astype(jnp.float32), dtab[...],
                               preferred_element_type=jnp.float32)
            lo = 48 + 64 * j
            acc += jnp.dot(dora_emb, wt[lo:lo + 64, :],
                           preferred_element_type=jnp.float32)
        acc += jnp.dot(h_emb, wt[368:384, :], preferred_element_type=jnp.float32)
        info = acc + ib[...]
        out_ref[...] = info

        for aref, tref in ((a48, t48_ref), (a49, t49_ref)):
            av = aref[...]
            oha = (av == lax.broadcasted_iota(jnp.int32, (BLK, TAB_PAD), 1))
            raw = jnp.dot(oha.astype(jnp.float32), tab_ref[...],
                          preferred_element_type=jnp.float32)
            tref[...] = jnp.where(av == SENTINEL, info, raw)


def _build_combined(tab_pad, scores, oya1, dsplit, hrs, a48, a49, ln_g, ln_b,
                    wst, sb, oyat, dtab, hwt, hb, wt, ib):
    nb = B // BLK
    full = lambda i: (0, 0)
    batch = lambda i: (jnp.maximum(i - 1, 0), 0)
    return pl.pallas_call(
        _tc_body,
        grid=(nb + 1,),
        in_specs=[
            pl.BlockSpec((TAB_PAD, D), full),
            pl.BlockSpec((BLK, 4), batch),
            pl.BlockSpec((BLK, 1), batch),
            pl.BlockSpec((BLK, 1), batch),
            pl.BlockSpec((BLK, 1), batch),
            pl.BlockSpec((BLK, 1), batch),
            pl.BlockSpec((BLK, 1), batch),
            pl.BlockSpec((BLK, 1), batch),
            pl.BlockSpec((BLK, 2), batch),
            pl.BlockSpec((BLK, 1), batch),
            pl.BlockSpec((BLK, 1), batch),
            pl.BlockSpec((1, 4), full),
            pl.BlockSpec((1, 4), full),
            pl.BlockSpec((4, 32), full),
            pl.BlockSpec((1, 32), full),
            pl.BlockSpec((4, 16), full),
            pl.BlockSpec((38, 64), full),
            pl.BlockSpec((2, 16), full),
            pl.BlockSpec((1, 16), full),
            pl.BlockSpec((384, D), full),
            pl.BlockSpec((1, D), full),
        ],
        out_specs=[
            pl.BlockSpec((BLK, D), lambda i: (i, 0)),
            pl.BlockSpec((BLK, D), batch),
            pl.BlockSpec((BLK, D), batch),
        ],
        out_shape=[
            jax.ShapeDtypeStruct((TAB_PAD + B, D), jnp.float32),
            jax.ShapeDtypeStruct((B, D), jnp.float32),
            jax.ShapeDtypeStruct((B, D), jnp.float32),
        ],
    )(tab_pad, scores, oya1, *dsplit, hrs, a48, a49, ln_g, ln_b,
      wst, sb, oyat, dtab, hwt, hb, wt, ib)


def _sc_gather(comb, act3d):
    mesh = plsc.VectorSubcoreMesh(core_axis_name="c", subcore_axis_name="s",
                                  num_cores=2, num_subcores=16)

    @functools.partial(
        pl.kernel,
        out_type=jax.ShapeDtypeStruct((B, S, D), jnp.float32),
        mesh=mesh,
        scratch_types=[
            pltpu.VMEM((RPW, S), jnp.int32),
            pltpu.VMEM((4, S, D), jnp.float32),
            pltpu.SemaphoreType.DMA,
            pltpu.SemaphoreType.DMA,
            pltpu.SemaphoreType.DMA,
            pltpu.SemaphoreType.DMA,
            pltpu.SemaphoreType.DMA,
            pltpu.SemaphoreType.DMA,
            pltpu.SemaphoreType.DMA,
            pltpu.SemaphoreType.DMA,
        ],
    )
    def k(comb_hbm, act_hbm, out_hbm, idx_v, bufs,
          g0, g1, g2, g3, s0, s1, s2, s3):
        gsem = (g0, g1, g2, g3)
        ssem = (s0, s1, s2, s3)
        nc = 2
        wid = lax.axis_index("s") * nc + lax.axis_index("c")
        b0 = wid * RPW
        pltpu.sync_copy(act_hbm.at[wid], idx_v)

        def fix(r, _):
            bsrc = b0 + r + TAB_PAD
            for off in (0, 16, 32):
                v = idx_v[r, pl.ds(off, 16)]
                idx_v[r, pl.ds(off, 16)] = jnp.where(v == SENTINEL, bsrc, v)
            return 0

        lax.fori_loop(0, RPW, fix, 0)

        def g(r, slot):
            pltpu.async_copy(comb_hbm.at[idx_v.at[r]], bufs.at[slot], gsem[slot])

        def wg(r, slot):
            pltpu.make_async_copy(
                comb_hbm.at[idx_v.at[r]], bufs.at[slot], gsem[slot]).wait()

        def s(r, slot):
            pltpu.async_copy(bufs.at[slot, pl.ds(0, SFULL)],
                             out_hbm.at[b0 + r, pl.ds(0, SFULL)], ssem[slot])

        def ws(r, slot):
            pltpu.make_async_copy(
                bufs.at[slot, pl.ds(0, SFULL)],
                out_hbm.at[b0 + r, pl.ds(0, SFULL)], ssem[slot]).wait()

        g(0, 0)
        g(1, 1)
        g(2, 2)
        wg(0, 0); s(0, 0); g(3, 3)
        wg(1, 1); s(1, 1); ws(0, 0); g(4, 0)
        wg(2, 2); s(2, 2); ws(1, 1); g(5, 1)
        wg(3, 3); s(3, 3); ws(2, 2); g(6, 2)

        def body(p, _):
            for q in range(4):
                r = 4 * p + q
                wg(r, q)
                s(r, q)
                ws(r - 1, (q + 3) % 4)
                g(r + 3, (q + 3) % 4)
            return 0

        lax.fori_loop(1, RPW // 4 - 1, body, 0)

        r = RPW - 4
        wg(r, 0); s(r, 0); ws(r - 1, 3); g(r + 3, 3)
        wg(r + 1, 1); s(r + 1, 1)
        wg(r + 2, 2); s(r + 2, 2)
        wg(r + 3, 3); s(r + 3, 3)
        ws(r, 0); ws(r + 1, 1); ws(r + 2, 2); ws(r + 3, 3)

    return k(comb, act3d)


def kernel(scores, oya, dora, honba_riichi_sticks, action, mask, action_table,
           info_W, info_b, ln_g, ln_b, scores_W, scores_b, oya_table,
           dora_table, hrs_W, hrs_b):
    del mask
    tab_pad = jnp.zeros((TAB_PAD, D), jnp.float32).at[:NTAB].set(action_table)
    oya1 = oya.astype(jnp.int32).reshape(B, 1)
    dora_i = dora.astype(jnp.int32)
    dsplit = [dora_i[:, j:j + 1] for j in range(5)]
    act_i = action.astype(jnp.int32)
    a48 = act_i[:, 48:49]
    a49 = act_i[:, 49:50]
    comb, t48, t49 = _build_combined(
        tab_pad, scores, oya1, dsplit, honba_riichi_sticks, a48, a49,
        ln_g.reshape(1, 4), ln_b.reshape(1, 4),
        scores_W.T, scores_b.reshape(1, 32),
        oya_table, dora_table,
        hrs_W.T, hrs_b.reshape(1, 16),
        info_W.T, info_b.reshape(1, D))
    act3d = act_i.reshape(NW, RPW, S)
    out = _sc_gather(comb, act3d)
    out = out.at[:, 48, :].set(t48)
    out = out.at[:, 49, :].set(t49)
    return out


# speedup vs baseline: 1.0441x; 1.0441x over previous
import functools

import jax
import jax.numpy as jnp
from jax import lax
from jax.experimental import pallas as pl
from jax.experimental.pallas import tpu as pltpu
from jax.experimental.pallas import tpu_sc as plsc

B = 4096
S = 50
D = 512
NTAB = 225
TAB_PAD = 256
SENTINEL = 224

BLK = 256
NW = 32
TOT = B * S
PER_W = TOT // NW
CHUNK = 64
NCHUNK = PER_W // CHUNK
IDX_MINOR = 64


def _tc_body(tab_ref, sc_ref, oy_ref, d0, d1, d2, d3, d4, hr_ref,
             lng, lnb, wst, sb, oyat, dtab, hwt, hb, wt, ib, out_ref):
    i = pl.program_id(0)

    @pl.when(i == 0)
    def _():
        out_ref[...] = tab_ref[...]

    @pl.when(i > 0)
    def _():
        x = sc_ref[...]
        mu = jnp.mean(x, axis=-1, keepdims=True)
        xc = x - mu
        var = jnp.mean(xc * xc, axis=-1, keepdims=True)
        xn = xc * lax.rsqrt(var + 1e-5) * lng[...] + lnb[...]
        s_emb = jnp.dot(xn, wst[...], preferred_element_type=jnp.float32) + sb[...]

        oh = (oy_ref[...] == lax.broadcasted_iota(jnp.int32, (BLK, 4), 1))
        oya_emb = jnp.dot(oh.astype(jnp.float32), oyat[...],
                          preferred_element_type=jnp.float32)

        h_emb = jnp.dot(hr_ref[...], hwt[...],
                        preferred_element_type=jnp.float32) + hb[...]

        acc = jnp.dot(s_emb, wt[0:32, :], preferred_element_type=jnp.float32)
        acc += jnp.dot(oya_emb, wt[32:48, :], preferred_element_type=jnp.float32)
        for j, dref in enumerate((d0, d1, d2, d3, d4)):
            ohd = (dref[...] == lax.broadcasted_iota(jnp.int32, (BLK, 38), 1))
            dora_emb = jnp.dot(ohd.astype(jnp.float32), dtab[...],
                               preferred_element_type=jnp.float32)
            lo = 48 + 64 * j
            acc += jnp.dot(dora_emb, wt[lo:lo + 64, :],
                           preferred_element_type=jnp.float32)
        acc += jnp.dot(h_emb, wt[368:384, :], preferred_element_type=jnp.float32)
        out_ref[...] = acc + ib[...]


def _build_combined(tab_pad, scores, oya1, dsplit, hrs, ln_g, ln_b,
                    wst, sb, oyat, dtab, hwt, hb, wt, ib):
    nb = B // BLK
    full = lambda i: (0, 0)
    batch = lambda i: (jnp.maximum(i - 1, 0), 0)
    return pl.pallas_call(
        _tc_body,
        grid=(nb + 1,),
        in_specs=[
            pl.BlockSpec((TAB_PAD, D), full),
            pl.BlockSpec((BLK, 4), batch),
            pl.BlockSpec((BLK, 1), batch),
            pl.BlockSpec((BLK, 1), batch),
            pl.BlockSpec((BLK, 1), batch),
            pl.BlockSpec((BLK, 1), batch),
            pl.BlockSpec((BLK, 1), batch),
            pl.BlockSpec((BLK, 1), batch),
            pl.BlockSpec((BLK, 2), batch),
            pl.BlockSpec((1, 4), full),
            pl.BlockSpec((1, 4), full),
            pl.BlockSpec((4, 32), full),
            pl.BlockSpec((1, 32), full),
            pl.BlockSpec((4, 16), full),
            pl.BlockSpec((38, 64), full),
            pl.BlockSpec((2, 16), full),
            pl.BlockSpec((1, 16), full),
            pl.BlockSpec((384, D), full),
            pl.BlockSpec((1, D), full),
        ],
        out_specs=pl.BlockSpec((BLK, D), lambda i: (i, 0)),
        out_shape=jax.ShapeDtypeStruct((TAB_PAD + B, D), jnp.float32),
    )(tab_pad, scores, oya1, *dsplit, hrs, ln_g, ln_b,
      wst, sb, oyat, dtab, hwt, hb, wt, ib)


def _sc_gather(comb, act2d):
    mesh = plsc.VectorSubcoreMesh(core_axis_name="c", subcore_axis_name="s",
                                  num_cores=2, num_subcores=16)

    @functools.partial(
        pl.kernel,
        out_type=jax.ShapeDtypeStruct((TOT, D), jnp.float32),
        mesh=mesh,
        scratch_types=[
            pltpu.VMEM((NCHUNK, CHUNK), jnp.int32),
            pltpu.VMEM((3, CHUNK, D), jnp.float32),
            pltpu.SemaphoreType.DMA,
            pltpu.SemaphoreType.DMA,
            pltpu.SemaphoreType.DMA,
            pltpu.SemaphoreType.DMA,
            pltpu.SemaphoreType.DMA,
            pltpu.SemaphoreType.DMA,
        ],
    )
    def k(comb_hbm, act_hbm, out_hbm, idx_v, bufs, g0, g1, g2, s0, s1, s2):
        gsem = (g0, g1, g2)
        ssem = (s0, s1, s2)
        nc = 2
        wid = lax.axis_index("s") * nc + lax.axis_index("c")
        out0 = wid * PER_W
        pltpu.sync_copy(act_hbm.at[wid], idx_v)

        lane = lax.iota(jnp.int32, 16)

        b0 = wid * (PER_W // S)

        def fix(j, _):
            for kk in range(IDX_MINOR // 16):
                v = idx_v[j, pl.ds(kk * 16, 16)]
                nloc = j * IDX_MINOR + kk * 16 + lane
                b = b0 + ((nloc * 5243) >> 18)
                idx_v[j, pl.ds(kk * 16, 16)] = jnp.where(
                    v == SENTINEL, b + TAB_PAD, v)
            return 0

        lax.fori_loop(0, NCHUNK, fix, 0)

        def g(c, slot):
            pltpu.async_copy(comb_hbm.at[idx_v.at[c]], bufs.at[slot], gsem[slot])

        def wg(c, slot):
            pltpu.make_async_copy(
                comb_hbm.at[idx_v.at[c]], bufs.at[slot], gsem[slot]).wait()

        def s(c, slot):
            pltpu.async_copy(bufs.at[slot],
                             out_hbm.at[pl.ds(out0 + c * CHUNK, CHUNK)],
                             ssem[slot])

        def ws(c, slot):
            pltpu.make_async_copy(
                bufs.at[slot],
                out_hbm.at[pl.ds(out0 + c * CHUNK, CHUNK)], ssem[slot]).wait()

        g(0, 0)
        g(1, 1)
        wg(0, 0); s(0, 0); g(2, 2)
        wg(1, 1); s(1, 1); ws(0, 0); g(3, 0)
        wg(2, 2); s(2, 2); ws(1, 1); g(4, 1)

        def body(p, _):
            for q in range(3):
                c = 3 * p + q
                wg(c, q)
                s(c, q)
                ws(c - 1, (q + 2) % 3)
                g(c + 2, (q + 2) % 3)
            return 0

        lax.fori_loop(1, (NCHUNK - 4) // 3, body, 0)

        for c in (NCHUNK - 4, NCHUNK - 3):
            wg(c, c % 3); s(c, c % 3); ws(c - 1, (c + 2) % 3); g(c + 2, (c + 2) % 3)
        c = NCHUNK - 2
        wg(c, c % 3); s(c, c % 3); ws(c - 1, (c - 1) % 3)
        c = NCHUNK - 1
        wg(c, c % 3); s(c, c % 3)
        ws(NCHUNK - 2, (NCHUNK - 2) % 3)
        ws(NCHUNK - 1, (NCHUNK - 1) % 3)

    return k(comb, act2d)


def kernel(scores, oya, dora, honba_riichi_sticks, action, mask, action_table,
           info_W, info_b, ln_g, ln_b, scores_W, scores_b, oya_table,
           dora_table, hrs_W, hrs_b):
    del mask
    tab_pad = jnp.zeros((TAB_PAD, D), jnp.float32).at[:NTAB].set(action_table)
    oya1 = oya.astype(jnp.int32).reshape(B, 1)
    dora_i = dora.astype(jnp.int32)
    dsplit = [dora_i[:, j:j + 1] for j in range(5)]
    comb = _build_combined(
        tab_pad, scores, oya1, dsplit, honba_riichi_sticks,
        ln_g.reshape(1, 4), ln_b.reshape(1, 4),
        scores_W.T, scores_b.reshape(1, 32),
        oya_table, dora_table,
        hrs_W.T, hrs_b.reshape(1, 16),
        info_W.T, info_b.reshape(1, D))
    act2d = action.astype(jnp.int32).reshape(NW, NCHUNK, IDX_MINOR)
    out2d = _sc_gather(comb, act2d)
    return out2d.reshape(B, S, D)
